# baseline (speedup 1.0000x reference)
import functools

import jax
import jax.numpy as jnp
from jax import lax
from jax.experimental import pallas as pl
from jax.experimental.pallas import tpu as pltpu
from jax.experimental.pallas import tpu_sc as plsc

D = 256
K = 8192
BN = 512
RB = 128
KT = 2048
SC_CHUNK = 128


def _dist_kernel(x_ref, e_ref, idx_ref, loss_ref, e2_ref, acc_ref):
    i = pl.program_id(0)
    ng = pl.num_programs(0)

    @pl.when(i == 0)
    def _():
        e2_ref[...] = jnp.sum(e_ref[...] * e_ref[...], axis=0, keepdims=True)
        acc_ref[...] = jnp.zeros_like(acc_ref)

    xm2 = x_ref[...] * -2.0
    xes = []
    for t in range(K // KT):
        e = e_ref[:, t * KT:(t + 1) * KT]
        xes.append(lax.dot_general(xm2, e, (((1,), (0,)), ((), ())),
                                   preferred_element_type=jnp.float32))

    lane = lax.broadcasted_iota(jnp.int32, (RB, 128), 1)
    for rb in range(BN // RB):
        r0 = rb * RB
        best = None
        bg = None
        for g in range(K // 128):
            t, j = divmod(g, KT // 128)
            d = (xes[t][r0:r0 + RB, j * 128:(j + 1) * 128]
                 + e2_ref[0, g * 128:(g + 1) * 128][None, :])
            gi = jnp.full((RB, 128), g, jnp.int32)
            if best is None:
                best, bg = d, gi
            else:
                m = d < best
                best = jnp.where(m, d, best)
                bg = jnp.where(m, gi, bg)

        col = bg * 128 + lane
        minv = jnp.min(best, axis=1, keepdims=True)
        idx = jnp.min(jnp.where(best == minv, col, K), axis=1)
        idx_ref[0, 0, r0:r0 + RB] = idx
        xr = x_ref[r0:r0 + RB, :]
        x2 = jnp.sum(xr * xr, axis=1, keepdims=True)
        acc_ref[r0:r0 + RB, :] += minv + x2

    @pl.when(i == ng - 1)
    def _():
        loss_ref[...] = jnp.sum(acc_ref[...], keepdims=True).reshape(1, 1)


def _make_sc_gather(n, d):
    info = plsc.get_sparse_core_info()
    nc, ns = info.num_cores, info.num_subcores
    nw = nc * ns
    per_w = n // nw
    assert per_w * nw == n
    chunk = next(c for c in range(SC_CHUNK, 0, -8) if per_w % c == 0)
    n_ch = per_w // chunk
    mesh = plsc.VectorSubcoreMesh(core_axis_name="c", subcore_axis_name="s")

    nbuf = min(3, n_ch)

    @functools.partial(
        pl.kernel, mesh=mesh,
        out_type=jax.ShapeDtypeStruct((n, d), jnp.float32),
        scratch_types=(
            [pltpu.VMEM((per_w,), jnp.int32)]
            + [pltpu.VMEM((chunk, d), jnp.float32) for _ in range(nbuf)]
            + [pltpu.SemaphoreType.DMA for _ in range(2 * nbuf)]
        ),
    )
    def gather_kernel(table_hbm, idx_hbm, out_hbm, idx_v, *bufs_sems):
        rows = bufs_sems[:nbuf]
        gsems = bufs_sems[nbuf:2 * nbuf]
        ssems = bufs_sems[2 * nbuf:]
        wid = lax.axis_index("s") * nc + lax.axis_index("c")
        pltpu.sync_copy(idx_hbm.at[pl.ds(wid * per_w, per_w)], idx_v)

        def fire(c):
            b = c % nbuf
            return pltpu.async_copy(
                table_hbm.at[idx_v.at[pl.ds(c * chunk, chunk)]],
                rows[b], gsems[b])

        gcp = [fire(c) for c in range(nbuf)]
        scp = [None] * nbuf
        for c in range(n_ch):
            b = c % nbuf
            gcp[b].wait()
            scp[b] = pltpu.async_copy(
                rows[b], out_hbm.at[pl.ds((wid * n_ch + c) * chunk, chunk)],
                ssems[b])
            if c + nbuf < n_ch:
                scp[b].wait()
                gcp[b] = fire(c + nbuf)
        for c in range(max(0, n_ch - nbuf), n_ch):
            scp[c % nbuf].wait()

    return gather_kernel


N_CHUNKS = 1


def _dist_call(xc, embedding):
    nc_rows = xc.shape[0]
    g = nc_rows // BN
    return pl.pallas_call(
        _dist_kernel,
        grid=(g,),
        in_specs=[
            pl.BlockSpec((BN, D), lambda i: (i, 0)),
            pl.BlockSpec((D, K), lambda i: (0, 0)),
        ],
        out_specs=[
            pl.BlockSpec((1, 1, BN), lambda i: (i, 0, 0)),
            pl.BlockSpec((1, 1), lambda i: (0, 0)),
        ],
        out_shape=[
            jax.ShapeDtypeStruct((g, 1, BN), jnp.int32),
            jax.ShapeDtypeStruct((1, 1), jnp.float32),
        ],
        scratch_shapes=[
            pltpu.VMEM((1, K), jnp.float32),
            pltpu.VMEM((BN, 1), jnp.float32),
        ],
    )(xc, embedding)


def kernel(x, embedding):
    n = x.shape[0]
    rows_c = n // N_CHUNKS
    table = embedding.T
    gather = _make_sc_gather(rows_c, D)
    qs, idxs, loss_sums = [], [], []
    for c in range(N_CHUNKS):
        xc = lax.slice_in_dim(x, c * rows_c, (c + 1) * rows_c, axis=0)
        idx3, ls = _dist_call(xc, embedding)
        idxc = idx3.reshape(rows_c)
        qs.append(gather(table, idxc))
        idxs.append(idxc)
        loss_sums.append(ls[0, 0])
    quantized = jnp.concatenate(qs, axis=0)
    idx = jnp.concatenate(idxs, axis=0)
    loss = sum(loss_sums) * (0.25 / (n * D))
    return quantized, loss, idx.reshape(n, 1)


# speedup vs baseline: 1.0332x; 1.0332x over previous
import functools

import jax
import jax.numpy as jnp
from jax import lax
from jax.experimental import pallas as pl
from jax.experimental.pallas import tpu as pltpu
from jax.experimental.pallas import tpu_sc as plsc

D = 256
K = 8192
BN = 1024
RB = 64
KT = 4096
SC_CHUNK = 128


def _dist_kernel(x_ref, e_ref, idx_ref, loss_ref, e2_ref, acc_ref):
    i = pl.program_id(0)
    ng = pl.num_programs(0)

    @pl.when(i == 0)
    def _():
        e2_ref[...] = jnp.sum(e_ref[...] * e_ref[...], axis=0, keepdims=True)
        acc_ref[...] = jnp.zeros_like(acc_ref)

    xm2 = x_ref[...] * -2.0
    xes = []
    for t in range(K // KT):
        e = e_ref[:, t * KT:(t + 1) * KT]
        xes.append(lax.dot_general(xm2, e, (((1,), (0,)), ((), ())),
                                   preferred_element_type=jnp.float32))

    lane = lax.broadcasted_iota(jnp.int32, (RB, 128), 1)
    for rb in range(BN // RB):
        r0 = rb * RB
        best = None
        bg = None
        for g in range(K // 128):
            t, j = divmod(g, KT // 128)
            d = (xes[t][r0:r0 + RB, j * 128:(j + 1) * 128]
                 + e2_ref[0, g * 128:(g + 1) * 128][None, :])
            gi = jnp.full((RB, 128), g, jnp.int32)
            if best is None:
                best, bg = d, gi
            else:
                m = d < best
                best = jnp.where(m, d, best)
                bg = jnp.where(m, gi, bg)

        col = bg * 128 + lane
        minv = jnp.min(best, axis=1, keepdims=True)
        idx = jnp.min(jnp.where(best == minv, col, K), axis=1)
        idx_ref[0, 0, r0:r0 + RB] = idx
        xr = x_ref[r0:r0 + RB, :]
        x2 = jnp.sum(xr * xr, axis=1, keepdims=True)
        acc_ref[r0:r0 + RB, :] += minv + x2

    @pl.when(i == ng - 1)
    def _():
        loss_ref[...] = jnp.sum(acc_ref[...], keepdims=True).reshape(1, 1)


def _make_sc_gather(n, d):
    info = plsc.get_sparse_core_info()
    nc, ns = info.num_cores, info.num_subcores
    nw = nc * ns
    per_w = n // nw
    assert per_w * nw == n
    chunk = next(c for c in range(SC_CHUNK, 0, -8) if per_w % c == 0)
    n_ch = per_w // chunk
    mesh = plsc.VectorSubcoreMesh(core_axis_name="c", subcore_axis_name="s")

    nbuf = min(3, n_ch)

    @functools.partial(
        pl.kernel, mesh=mesh,
        out_type=jax.ShapeDtypeStruct((n, d), jnp.float32),
        scratch_types=(
            [pltpu.VMEM((per_w,), jnp.int32)]
            + [pltpu.VMEM((chunk, d), jnp.float32) for _ in range(nbuf)]
            + [pltpu.SemaphoreType.DMA for _ in range(2 * nbuf)]
        ),
    )
    def gather_kernel(table_hbm, idx_hbm, out_hbm, idx_v, *bufs_sems):
        rows = bufs_sems[:nbuf]
        gsems = bufs_sems[nbuf:2 * nbuf]
        ssems = bufs_sems[2 * nbuf:]
        wid = lax.axis_index("s") * nc + lax.axis_index("c")
        pltpu.sync_copy(idx_hbm.at[pl.ds(wid * per_w, per_w)], idx_v)

        def fire(c):
            b = c % nbuf
            return pltpu.async_copy(
                table_hbm.at[idx_v.at[pl.ds(c * chunk, chunk)]],
                rows[b], gsems[b])

        gcp = [fire(c) for c in range(nbuf)]
        scp = [None] * nbuf
        for c in range(n_ch):
            b = c % nbuf
            gcp[b].wait()
            scp[b] = pltpu.async_copy(
                rows[b], out_hbm.at[pl.ds((wid * n_ch + c) * chunk, chunk)],
                ssems[b])
            if c + nbuf < n_ch:
                scp[b].wait()
                gcp[b] = fire(c + nbuf)
        for c in range(max(0, n_ch - nbuf), n_ch):
            scp[c % nbuf].wait()

    return gather_kernel


N_CHUNKS = 1


def _dist_call(xc, embedding):
    nc_rows = xc.shape[0]
    g = nc_rows // BN
    return pl.pallas_call(
        _dist_kernel,
        grid=(g,),
        in_specs=[
            pl.BlockSpec((BN, D), lambda i: (i, 0)),
            pl.BlockSpec((D, K), lambda i: (0, 0)),
        ],
        out_specs=[
            pl.BlockSpec((1, 1, BN), lambda i: (i, 0, 0)),
            pl.BlockSpec((1, 1), lambda i: (0, 0)),
        ],
        out_shape=[
            jax.ShapeDtypeStruct((g, 1, BN), jnp.int32),
            jax.ShapeDtypeStruct((1, 1), jnp.float32),
        ],
        scratch_shapes=[
            pltpu.VMEM((1, K), jnp.float32),
            pltpu.VMEM((BN, 1), jnp.float32),
        ],
    )(xc, embedding)


def kernel(x, embedding):
    n = x.shape[0]
    rows_c = n // N_CHUNKS
    table = embedding.T
    gather = _make_sc_gather(rows_c, D)
    qs, idxs, loss_sums = [], [], []
    for c in range(N_CHUNKS):
        xc = lax.slice_in_dim(x, c * rows_c, (c + 1) * rows_c, axis=0)
        idx3, ls = _dist_call(xc, embedding)
        idxc = idx3.reshape(rows_c)
        qs.append(gather(table, idxc))
        idxs.append(idxc)
        loss_sums.append(ls[0, 0])
    quantized = jnp.concatenate(qs, axis=0)
    idx = jnp.concatenate(idxs, axis=0)
    loss = sum(loss_sums) * (0.25 / (n * D))
    return quantized, loss, idx.reshape(n, 1)
